# Initial kernel scaffold; baseline (speedup 1.0000x reference)
import jax
import jax.numpy as jnp
from jax.experimental import pallas as pl


def kernel(features, transitions, lengths, tags):
    raise NotImplementedError("write your pallas kernel here")


# speedup vs baseline: 6.1916x; 6.1916x over previous
import functools

import jax
import jax.numpy as jnp
from jax.experimental import pallas as pl
from jax.experimental.pallas import tpu as pltpu

_START = 126
_STOP = 127


def _crf_kernel(ft_ref, f2_ref, tg2_ref, ps_ref, pstop_ref, tmask_ref,
                emask_ref, len_ref, trans_ref, out_ref, es_ref):
    T, B, K = ft_ref.shape
    f32 = jnp.float32

    rows = f2_ref.shape[0]
    iota_k = jax.lax.broadcasted_iota(jnp.int32, (rows, K), 1)
    onehot_e = (iota_k == tg2_ref[...]).astype(f32)
    gold_emit = jnp.sum(f2_ref[...] * onehot_e * emask_ref[...])

    rows_t = ps_ref.shape[0]
    iota_t = jax.lax.broadcasted_iota(jnp.int32, (rows_t, K), 1)
    ostart = (iota_t == ps_ref[...]).astype(jnp.bfloat16)
    ostop = (iota_t == pstop_ref[...]).astype(jnp.bfloat16)
    om = ostart * tmask_ref[...].astype(jnp.bfloat16)
    cmat = jax.lax.dot_general(
        ostop, om, (((0,), (0,)), ((), ())),
        preferred_element_type=f32)
    gold_trans = jnp.sum(cmat * trans_ref[...])

    es_ref[...] = jnp.exp(ft_ref[...])
    exp_t = jnp.exp(trans_ref[...]).T
    exp_stop = jnp.exp(trans_ref[_STOP, :]).reshape(1, K)

    iota_b = jax.lax.broadcasted_iota(jnp.int32, (B, K), 1)
    u0 = (iota_b == _START).astype(f32)
    c0 = jnp.zeros((B, 1), f32)
    lens = len_ref[...]

    def step(t, carry):
        u, c, rcap, ccap = carry
        r = jnp.sum(u * exp_stop, axis=1, keepdims=True)
        hit = lens == t
        rcap = jnp.where(hit, r, rcap)
        ccap = jnp.where(hit, c, ccap)
        v = jax.lax.dot_general(
            u, exp_t, (((1,), (0,)), ((), ())),
            preferred_element_type=f32,
            precision=jax.lax.Precision.HIGHEST)
        u = es_ref[t] * v
        m = jnp.max(u, axis=1, keepdims=True)
        u = u * (1.0 / m)
        c = c + jnp.log(m)
        return u, c, rcap, ccap

    _, _, rcap, ccap = jax.lax.fori_loop(
        0, T, step, (u0, c0, jnp.ones((B, 1), f32), c0))

    fwd = jnp.log(rcap) + ccap
    loss = (jnp.sum(fwd) - gold_emit - gold_trans) / B
    out_ref[...] = jnp.reshape(loss, (1, 1))


@functools.partial(jax.jit, static_argnames=())
def kernel(features, transitions, lengths, tags):
    B, T, K = features.shape
    i32 = jnp.int32
    tags = tags.astype(i32)
    lengths = lengths.astype(i32)

    ft = jnp.transpose(features, (1, 0, 2))
    f2 = features.reshape(B * T, K)
    tg2 = tags.reshape(B * T, 1)

    pos = jnp.arange(T + 1, dtype=i32)[None, :]
    pad_start = jnp.concatenate(
        [jnp.full((B, 1), _START, i32), tags], axis=1)
    pad_stop = jnp.concatenate(
        [tags, jnp.full((B, 1), _STOP, i32)], axis=1)
    pad_stop = jnp.where(pos >= lengths[:, None], _STOP, pad_stop)
    tmask = (pos <= lengths[:, None]).astype(jnp.float32)
    emask = (jnp.arange(T, dtype=i32)[None, :]
             < lengths[:, None]).astype(jnp.float32)

    ps_flat = pad_start.reshape(-1, 1)
    pstop_flat = pad_stop.reshape(-1, 1)
    tmask_flat = tmask.reshape(-1, 1)
    emask_flat = emask.reshape(-1, 1)
    len2 = lengths.reshape(B, 1)

    out = pl.pallas_call(
        _crf_kernel,
        out_shape=jax.ShapeDtypeStruct((1, 1), jnp.float32),
        scratch_shapes=[pltpu.VMEM((T, B, K), jnp.float32)],
    )(ft, f2, tg2, ps_flat, pstop_flat, tmask_flat, emask_flat, len2,
      transitions)
    return out.reshape(())


# speedup vs baseline: 10.9544x; 1.7693x over previous
import functools

import jax
import jax.numpy as jnp
from jax.experimental import pallas as pl
from jax.experimental.pallas import tpu as pltpu

_START = 126
_STOP = 127


def _crf_kernel(ft_ref, f2_ref, tg2_ref, ps_ref, pstop_ref, tmask_ref,
                emask_ref, len_ref, trans_ref, out_ref, es_ref):
    T, B, K = ft_ref.shape
    f32 = jnp.float32

    rows = f2_ref.shape[0]
    iota_k = jax.lax.broadcasted_iota(jnp.int32, (rows, K), 1)
    onehot_e = (iota_k == tg2_ref[...]).astype(f32)
    gold_emit = jnp.sum(f2_ref[...] * onehot_e * emask_ref[...])

    rows_t = ps_ref.shape[0]
    iota_t = jax.lax.broadcasted_iota(jnp.int32, (rows_t, K), 1)
    ostart = (iota_t == ps_ref[...]).astype(jnp.bfloat16)
    ostop = (iota_t == pstop_ref[...]).astype(jnp.bfloat16)
    om = ostart * tmask_ref[...].astype(jnp.bfloat16)
    cmat = jax.lax.dot_general(
        ostop, om, (((0,), (0,)), ((), ())),
        preferred_element_type=f32)
    gold_trans = jnp.sum(cmat * trans_ref[...])

    es_ref[...] = jnp.exp(ft_ref[...])
    exp_t = jnp.exp(trans_ref[...]).T.astype(jnp.bfloat16)

    iota_b = jax.lax.broadcasted_iota(jnp.int32, (B, K), 1)
    u0 = (iota_b == _START).astype(f32)
    c0 = jnp.zeros((B, 1), f32)
    lens = len_ref[...]

    UNROLL = 4

    def step(blk, carry):
        u, c, rcap, ccap = carry
        for k in range(UNROLL):
            t = blk * UNROLL + k
            v = jax.lax.dot_general(
                u.astype(jnp.bfloat16), exp_t, (((1,), (0,)), ((), ())),
                preferred_element_type=f32)
            hit = lens == t
            rcap = jnp.where(hit, v[:, _STOP:_STOP + 1], rcap)
            ccap = jnp.where(hit, c, ccap)
            u = es_ref[t] * v
        m = jnp.max(u, axis=1, keepdims=True)
        u = u * (1.0 / m)
        c = c + jnp.log(m)
        return u, c, rcap, ccap

    _, _, rcap, ccap = jax.lax.fori_loop(
        0, T // UNROLL, step, (u0, c0, jnp.ones((B, 1), f32), c0))

    fwd = jnp.log(rcap) + ccap
    loss = (jnp.sum(fwd) - gold_emit - gold_trans) / B
    out_ref[...] = jnp.reshape(loss, (1, 1))


@functools.partial(jax.jit, static_argnames=())
def kernel(features, transitions, lengths, tags):
    B, T, K = features.shape
    i32 = jnp.int32
    tags = tags.astype(i32)
    lengths = lengths.astype(i32)

    ft = jnp.transpose(features, (1, 0, 2))
    f2 = features.reshape(B * T, K)
    tg2 = tags.reshape(B * T, 1)

    pos = jnp.arange(T + 1, dtype=i32)[None, :]
    pad_start = jnp.concatenate(
        [jnp.full((B, 1), _START, i32), tags], axis=1)
    pad_stop = jnp.concatenate(
        [tags, jnp.full((B, 1), _STOP, i32)], axis=1)
    pad_stop = jnp.where(pos >= lengths[:, None], _STOP, pad_stop)
    tmask = (pos <= lengths[:, None]).astype(jnp.float32)
    emask = (jnp.arange(T, dtype=i32)[None, :]
             < lengths[:, None]).astype(jnp.float32)

    ps_flat = pad_start.reshape(-1, 1)
    pstop_flat = pad_stop.reshape(-1, 1)
    tmask_flat = tmask.reshape(-1, 1)
    emask_flat = emask.reshape(-1, 1)
    len2 = lengths.reshape(B, 1)

    out = pl.pallas_call(
        _crf_kernel,
        out_shape=jax.ShapeDtypeStruct((1, 1), jnp.float32),
        scratch_shapes=[pltpu.VMEM((T, B, K), jnp.float32)],
    )(ft, f2, tg2, ps_flat, pstop_flat, tmask_flat, emask_flat, len2,
      transitions)
    return out.reshape(())
